# baseline (speedup 1.0000x reference)
import functools

import jax
import jax.numpy as jnp
from jax.experimental import pallas as pl

TOPK = 4
GAMMA = 0.1
BB = 256


def _agl_kernel(x_ref, pose_ref, w_ref, b_ref, eye_ref, h_ref, adj_ref,
                *, bb, k, din, dout):
    r = bb * k
    x2 = x_ref[...]
    h2 = jnp.dot(x2, w_ref[...],
                 preferred_element_type=jnp.float32) + b_ref[...]
    h_ref[...] = h2

    ones_col = jnp.ones((r, 1), dtype=jnp.float32)
    sq2 = jnp.dot(h2 * h2, jnp.ones((din, 1), dtype=jnp.float32),
                  preferred_element_type=jnp.float32)
    a2 = jnp.concatenate([ones_col, h2], axis=1)
    b2 = jnp.concatenate([sq2, -2.0 * h2], axis=1)
    a3 = a2.reshape(bb, k, din + 1)
    b3 = b2.reshape(bb, k, din + 1)
    key3 = jax.lax.dot_general(
        a3, b3, (((2,), (2,)), ((0,), (0,))),
        preferred_element_type=jnp.float32,
    )
    key_t = key3.reshape(r, k).T

    work = key_t
    for _ in range(TOPK - 1):
        m = jnp.min(work, axis=0, keepdims=True)
        work = jnp.where(work <= m, jnp.float32(jnp.inf), work)
    thresh = jnp.min(work, axis=0, keepdims=True)

    c_knn = jnp.float32(GAMMA / (1.0 + GAMMA))
    knn = jnp.where(key_t <= thresh, c_knn, jnp.float32(0.0))
    blended = (pose_ref[...].T * jnp.float32(1.0 / (1.0 + GAMMA))
               + knn + eye_ref[...])
    norm = jnp.maximum(jnp.sum(jnp.abs(blended), axis=0, keepdims=True),
                       1e-12)
    adj_ref[...] = (blended * (1.0 / norm)).T


@jax.jit
def kernel(x, pose_adj, W, b):
    B, K, DIN = x.shape
    DOUT = W.shape[1]
    R = BB * K
    grid = (B // BB,)
    x2 = x.reshape(B * K, DIN)
    pose2 = pose_adj.reshape(B * K, K)
    b2 = b.reshape(1, DOUT)
    col = jnp.arange(R, dtype=jnp.int32) % K
    eye_t = ((jnp.arange(K, dtype=jnp.int32)[:, None] == col[None, :])
             .astype(jnp.float32) * (GAMMA / (1.0 + GAMMA)))

    h2, adj2 = pl.pallas_call(
        functools.partial(_agl_kernel, bb=BB, k=K, din=DIN, dout=DOUT),
        grid=grid,
        in_specs=[
            pl.BlockSpec((R, DIN), lambda i: (i, 0)),
            pl.BlockSpec((R, K), lambda i: (i, 0)),
            pl.BlockSpec((DIN, DOUT), lambda i: (0, 0)),
            pl.BlockSpec((1, DOUT), lambda i: (0, 0)),
            pl.BlockSpec((K, R), lambda i: (0, 0)),
        ],
        out_specs=[
            pl.BlockSpec((R, DOUT), lambda i: (i, 0)),
            pl.BlockSpec((R, K), lambda i: (i, 0)),
        ],
        out_shape=[
            jax.ShapeDtypeStruct((B * K, DOUT), jnp.float32),
            jax.ShapeDtypeStruct((B * K, K), jnp.float32),
        ],
    )(x2, pose2, W, b2, eye_t)
    return (h2.reshape(B, K, DOUT), adj2.reshape(B, K, K))


# speedup vs baseline: 1.2756x; 1.2756x over previous
import functools

import jax
import jax.numpy as jnp
from jax.experimental import pallas as pl

TOPK = 4
GAMMA = 0.1
BB = 256


def _agl_kernel(x_ref, pose_ref, w_ref, b_ref, h_ref, adj_ref,
                *, bb, k, din, dout):
    r = bb * k
    x2 = x_ref[...]
    h2 = jnp.dot(x2, w_ref[...],
                 preferred_element_type=jnp.float32) + b_ref[...]
    h_ref[...] = h2

    h3 = h2.reshape(bb, k, dout)
    sq = jnp.sum(h3 * h3, axis=2)
    g = jax.lax.dot_general(
        h3, h3, (((2,), (2,)), ((0,), (0,))),
        preferred_element_type=jnp.float32,
    )
    d2 = sq[:, :, None] + sq[:, None, :] - 2.0 * g
    d2 = jnp.maximum(d2, 1e-12)

    work = d2
    for _ in range(TOPK - 1):
        m = jnp.min(work, axis=2, keepdims=True)
        work = jnp.where(work <= m, jnp.float32(jnp.inf), work)
    thresh = jnp.min(work, axis=2, keepdims=True)
    knn = (d2 <= thresh).astype(jnp.float32)

    col = jax.lax.broadcasted_iota(jnp.int32, (1, 1, k), 2)
    row = jax.lax.broadcasted_iota(jnp.int32, (1, k, k), 1)
    eye = (row == col).astype(jnp.float32)
    blended = (pose_ref[...] + GAMMA * (knn + eye)) / (1.0 + GAMMA)
    norm = jnp.maximum(jnp.sum(jnp.abs(blended), axis=2, keepdims=True),
                       1e-12)
    adj_ref[...] = blended / norm


@jax.jit
def kernel(x, pose_adj, W, b):
    B, K, DIN = x.shape
    DOUT = W.shape[1]
    R = BB * K
    grid = (B // BB,)
    x2 = x.reshape(B * K, DIN)
    b2 = b.reshape(1, DOUT)

    h2, adj = pl.pallas_call(
        functools.partial(_agl_kernel, bb=BB, k=K, din=DIN, dout=DOUT),
        grid=grid,
        in_specs=[
            pl.BlockSpec((R, DIN), lambda i: (i, 0)),
            pl.BlockSpec((BB, K, K), lambda i: (i, 0, 0)),
            pl.BlockSpec((DIN, DOUT), lambda i: (0, 0)),
            pl.BlockSpec((1, DOUT), lambda i: (0, 0)),
        ],
        out_specs=[
            pl.BlockSpec((R, DOUT), lambda i: (i, 0)),
            pl.BlockSpec((BB, K, K), lambda i: (i, 0, 0)),
        ],
        out_shape=[
            jax.ShapeDtypeStruct((B * K, DOUT), jnp.float32),
            jax.ShapeDtypeStruct((B, K, K), jnp.float32),
        ],
    )(x2, pose_adj, W, b2)
    return (h2.reshape(B, K, DOUT), adj)


# speedup vs baseline: 1.3420x; 1.0520x over previous
import functools

import jax
import jax.numpy as jnp
from jax.experimental import pallas as pl

TOPK = 4
GAMMA = 0.1
BB = 256


def _agl_kernel(x_ref, pose_ref, w_ref, b_ref, h_ref, adj_ref,
                *, bb, k, din, dout):
    xb = x_ref[...]
    h3 = jax.lax.dot_general(
        xb, w_ref[...], (((2,), (0,)), ((), ())),
        preferred_element_type=jnp.float32,
    ) + b_ref[...][None]
    h_ref[...] = h3

    sq = jnp.sum(h3 * h3, axis=2)
    g = jax.lax.dot_general(
        h3, h3, (((2,), (2,)), ((0,), (0,))),
        preferred_element_type=jnp.float32,
    )
    d2 = sq[:, :, None] + sq[:, None, :] - 2.0 * g

    work = d2
    for _ in range(TOPK - 1):
        m = jnp.min(work, axis=2, keepdims=True)
        work = jnp.where(work <= m, jnp.float32(jnp.inf), work)
    thresh = jnp.min(work, axis=2, keepdims=True)

    col = jax.lax.broadcasted_iota(jnp.int32, (1, 1, k), 2)
    row = jax.lax.broadcasted_iota(jnp.int32, (1, k, k), 1)
    eye_g = jnp.where(row == col, jnp.float32(GAMMA), jnp.float32(0.0))
    knn_g = jnp.where(d2 <= thresh, jnp.float32(GAMMA), jnp.float32(0.0))
    t = pose_ref[...] + (knn_g + eye_g)
    norm = jnp.sum(t, axis=2, keepdims=True)
    adj_ref[...] = t / norm


@jax.jit
def kernel(x, pose_adj, W, b):
    B, K, DIN = x.shape
    DOUT = W.shape[1]
    grid = (B // BB,)
    b2 = b.reshape(1, DOUT)

    h, adj = pl.pallas_call(
        functools.partial(_agl_kernel, bb=BB, k=K, din=DIN, dout=DOUT),
        grid=grid,
        in_specs=[
            pl.BlockSpec((BB, K, DIN), lambda i: (i, 0, 0)),
            pl.BlockSpec((BB, K, K), lambda i: (i, 0, 0)),
            pl.BlockSpec((DIN, DOUT), lambda i: (0, 0)),
            pl.BlockSpec((1, DOUT), lambda i: (0, 0)),
        ],
        out_specs=[
            pl.BlockSpec((BB, K, DOUT), lambda i: (i, 0, 0)),
            pl.BlockSpec((BB, K, K), lambda i: (i, 0, 0)),
        ],
        out_shape=[
            jax.ShapeDtypeStruct((B, K, DOUT), jnp.float32),
            jax.ShapeDtypeStruct((B, K, K), jnp.float32),
        ],
    )(x, pose_adj, W, b2)
    return (h, adj)


# speedup vs baseline: 1.3421x; 1.0001x over previous
import functools

import jax
import jax.numpy as jnp
from jax.experimental import pallas as pl
from jax.experimental.pallas import tpu as pltpu

TOPK = 4
GAMMA = 0.1
BB = 256


def _agl_kernel(x_ref, pose_ref, w_ref, b_ref, h_ref, adj_ref,
                *, bb, k, din, dout):
    xb = x_ref[...]
    h3 = jax.lax.dot_general(
        xb, w_ref[...], (((2,), (0,)), ((), ())),
        preferred_element_type=jnp.float32,
    ) + b_ref[...][None]
    h_ref[...] = h3

    sq = jnp.sum(h3 * h3, axis=2)
    g = jax.lax.dot_general(
        h3, h3, (((2,), (2,)), ((0,), (0,))),
        preferred_element_type=jnp.float32,
    )
    d2 = sq[:, :, None] + sq[:, None, :] - 2.0 * g

    work = d2
    for _ in range(TOPK - 1):
        m = jnp.min(work, axis=2, keepdims=True)
        work = jnp.where(work <= m, jnp.float32(jnp.inf), work)
    thresh = jnp.min(work, axis=2, keepdims=True)

    col = jax.lax.broadcasted_iota(jnp.int32, (1, 1, k), 2)
    row = jax.lax.broadcasted_iota(jnp.int32, (1, k, k), 1)
    eye_g = jnp.where(row == col, jnp.float32(GAMMA), jnp.float32(0.0))
    knn_g = jnp.where(d2 <= thresh, jnp.float32(GAMMA), jnp.float32(0.0))
    t = pose_ref[...] + (knn_g + eye_g)
    norm = jnp.sum(t, axis=2, keepdims=True)
    adj_ref[...] = t / norm


@jax.jit
def kernel(x, pose_adj, W, b):
    B, K, DIN = x.shape
    DOUT = W.shape[1]
    grid = (B // BB,)
    b2 = b.reshape(1, DOUT)

    h, adj = pl.pallas_call(
        functools.partial(_agl_kernel, bb=BB, k=K, din=DIN, dout=DOUT),
        grid=grid,
        compiler_params=pltpu.CompilerParams(
            dimension_semantics=("parallel",),
        ),
        in_specs=[
            pl.BlockSpec((BB, K, DIN), lambda i: (i, 0, 0)),
            pl.BlockSpec((BB, K, K), lambda i: (i, 0, 0)),
            pl.BlockSpec((DIN, DOUT), lambda i: (0, 0)),
            pl.BlockSpec((1, DOUT), lambda i: (0, 0)),
        ],
        out_specs=[
            pl.BlockSpec((BB, K, DOUT), lambda i: (i, 0, 0)),
            pl.BlockSpec((BB, K, K), lambda i: (i, 0, 0)),
        ],
        out_shape=[
            jax.ShapeDtypeStruct((B, K, DOUT), jnp.float32),
            jax.ShapeDtypeStruct((B, K, K), jnp.float32),
        ],
    )(x, pose_adj, W, b2)
    return (h, adj)
